# baseline (speedup 1.0000x reference)
import functools

import jax
import jax.numpy as jnp
from jax.experimental import pallas as pl
from jax.experimental.pallas import tpu as pltpu

_B, _N, _K = 4, 512, 8
_HID = 64
_JT = 128


def _edge_predictor_kernel(p_ref, pT_ref, W1_ref, b1_ref, W2_ref, b2_ref,
                           W3_ref, b3_ref, Wm1_ref, bm1_ref, Wm2T_ref,
                           bm2_ref, out_ref):
    N = _N
    f32 = jnp.float32

    p = p_ref[0]
    pT = pT_ref[0]
    px_c = p[:, 0:1]
    py_c = p[:, 1:2]
    px_r = pT[0:1, :]
    py_r = pT[1:2, :]
    dx = px_c - px_r
    dy = py_c - py_r
    d2 = dx * dx + dy * dy

    col = jax.lax.broadcasted_iota(jnp.int32, (N, N), 1)
    row = jax.lax.broadcasted_iota(jnp.int32, (N, N), 0)

    work = d2
    M = jnp.zeros((N, N), f32)
    for t in range(_K + 1):
        mv = jnp.min(work, axis=1, keepdims=True)
        idx = jnp.min(jnp.where(work == mv, col, N), axis=1,
                      keepdims=True)
        first = col == idx
        if t > 0:
            M = M + first.astype(f32)
        work = jnp.where(first, jnp.float32(jnp.inf), work)

    eye = (row == col).astype(f32)
    cnt = M + M.T + eye
    deg = jnp.sum(cnt, axis=0, keepdims=True)
    dis_r = 1.0 / jnp.sqrt(deg)
    dis_c = dis_r.T
    Anorm = cnt * dis_c * dis_r

    dot = functools.partial(jnp.dot, preferred_element_type=f32)

    XW = px_c * W1_ref[0:1, :] + py_c * W1_ref[1:2, :]
    h = jnp.maximum(dot(Anorm, XW) + b1_ref[0:1, :], 0.0)
    h = jnp.maximum(dot(Anorm, dot(h, W2_ref[...])) + b2_ref[0:1, :], 0.0)
    h = jnp.maximum(dot(Anorm, dot(h, W3_ref[...])) + b3_ref[0:1, :], 0.0)

    Wm1 = Wm1_ref[...]
    Ae = dot(h, Wm1[0:_HID, :]) + bm1_ref[0:1, :]
    Be = dot(h, Wm1[_HID:, :])
    BeT = Be.T
    w_col = Wm2T_ref[0:1, :].T
    c_col = dot(Ae, w_col)
    r_row = dot(Be, w_col).T
    base = bm2_ref[0:1, 0:1] * 2.0

    T = _JT
    tri = jax.lax.broadcasted_iota(jnp.int32, (T, T), 0) < \
        jax.lax.broadcasted_iota(jnp.int32, (T, T), 1)
    for it in range(N // T):
        i0 = it * T
        for jt in range(it, N // T):
            j0 = jt * T
            be = BeT[:, j0:j0 + T]
            acc = c_col[i0:i0 + T, :] + r_row[:, j0:j0 + T] + base
            for k in range(_HID):
                t = Ae[i0:i0 + T, k:k + 1] + be[k:k + 1, :]
                acc = acc + jnp.abs(t) * Wm2T_ref[0:1, k:k + 1]
            prob = jax.nn.sigmoid(acc * 0.5)
            if it == jt:
                up = jnp.where(tri, prob, 0.0)
                out_ref[0, i0:i0 + T, j0:j0 + T] = up + up.T
            else:
                out_ref[0, i0:i0 + T, j0:j0 + T] = prob
                out_ref[0, j0:j0 + T, i0:i0 + T] = prob.T


def kernel(batch_points, W1, b1, W2, b2, W3, b3, Wm1, bm1, Wm2, bm2):
    B, N = _B, _N
    pT = jnp.transpose(batch_points, (0, 2, 1))
    b1r = b1.reshape(1, _HID)
    b2r = b2.reshape(1, _HID)
    b3r = b3.reshape(1, _HID)
    bm1r = bm1.reshape(1, _HID)
    Wm2T = Wm2.reshape(1, _HID)
    bm2r = bm2.reshape(1, 1)

    rep = lambda shape: pl.BlockSpec(shape, lambda g: (0,) * len(shape))
    out = pl.pallas_call(
        _edge_predictor_kernel,
        grid=(B,),
        in_specs=[
            pl.BlockSpec((1, N, 2), lambda g: (g, 0, 0)),
            pl.BlockSpec((1, 2, N), lambda g: (g, 0, 0)),
            rep((2, _HID)), rep((1, _HID)),
            rep((_HID, _HID)), rep((1, _HID)),
            rep((_HID, _HID)), rep((1, _HID)),
            rep((2 * _HID, _HID)), rep((1, _HID)),
            rep((1, _HID)), rep((1, 1)),
        ],
        out_specs=pl.BlockSpec((1, N, N), lambda g: (g, 0, 0)),
        out_shape=jax.ShapeDtypeStruct((B, N, N), jnp.float32),
        compiler_params=pltpu.CompilerParams(
            dimension_semantics=("parallel",)),
    )(batch_points, pT, W1, b1r, W2, b2r, W3, b3r, Wm1, bm1r, Wm2T, bm2r)
    return out


# speedup vs baseline: 1.0474x; 1.0474x over previous
import functools

import jax
import jax.numpy as jnp
from jax.experimental import pallas as pl
from jax.experimental.pallas import tpu as pltpu

_B, _N, _K = 4, 512, 8
_HID = 64
_JT = 128


def _edge_predictor_kernel(p_ref, pT_ref, W1_ref, b1_ref, W2_ref, b2_ref,
                           W3_ref, b3_ref, Wm1_ref, bm1_ref, Wm2T_ref,
                           bm2_ref, out_ref):
    N = _N
    f32 = jnp.float32

    p = p_ref[0]
    pT = pT_ref[0]
    px_c = p[:, 0:1]
    py_c = p[:, 1:2]
    px_r = pT[0:1, :]
    py_r = pT[1:2, :]
    dx = px_c - px_r
    dy = py_c - py_r
    d2 = dx * dx + dy * dy

    col = jax.lax.broadcasted_iota(jnp.int32, (N, N), 1)
    row = jax.lax.broadcasted_iota(jnp.int32, (N, N), 0)

    work = d2
    M = jnp.zeros((N, N), f32)
    for t in range(_K + 1):
        mv = jnp.min(work, axis=1, keepdims=True)
        idx = jnp.min(jnp.where(work == mv, col, N), axis=1,
                      keepdims=True)
        first = col == idx
        if t > 0:
            M = M + first.astype(f32)
        work = jnp.where(first, jnp.float32(jnp.inf), work)

    eye = (row == col).astype(f32)
    cnt = M + M.T + eye
    deg = jnp.sum(cnt, axis=0, keepdims=True)
    dis_r = 1.0 / jnp.sqrt(deg)
    dis_c = dis_r.T
    Anorm = cnt * dis_c * dis_r

    dot = functools.partial(jnp.dot, preferred_element_type=f32)

    XW = px_c * W1_ref[0:1, :] + py_c * W1_ref[1:2, :]
    h = jnp.maximum(dot(Anorm, XW) + b1_ref[0:1, :], 0.0)
    h = jnp.maximum(dot(Anorm, dot(h, W2_ref[...])) + b2_ref[0:1, :], 0.0)
    h = jnp.maximum(dot(Anorm, dot(h, W3_ref[...])) + b3_ref[0:1, :], 0.0)

    Wm1 = Wm1_ref[...]
    Ae = dot(h, Wm1[0:_HID, :]) + bm1_ref[0:1, :]
    Be = dot(h, Wm1[_HID:, :])
    BeT = Be.T
    w_col = Wm2T_ref[0:1, :].T
    c_col = dot(Ae, w_col)
    r_row = dot(Be, w_col).T
    base = bm2_ref[0:1, 0:1] * 2.0

    T = _JT
    tri = jax.lax.broadcasted_iota(jnp.int32, (T, T), 0) < \
        jax.lax.broadcasted_iota(jnp.int32, (T, T), 1)
    for it in range(N // T):
        i0 = it * T
        jw = N - i0
        acc = c_col[i0:i0 + T, :] + r_row[:, i0:] + base
        for k in range(_HID):
            t = Ae[i0:i0 + T, k:k + 1] + BeT[k:k + 1, i0:]
            acc = acc + jnp.abs(t) * Wm2T_ref[0:1, k:k + 1]
        prob = jax.nn.sigmoid(acc * 0.5)
        up = jnp.where(tri, prob[:, 0:T], 0.0)
        out_ref[0, i0:i0 + T, i0:i0 + T] = up + up.T
        for jt in range(it + 1, N // T):
            off = jt * T - i0
            blk = prob[:, off:off + T]
            out_ref[0, i0:i0 + T, jt * T:(jt + 1) * T] = blk
            out_ref[0, jt * T:(jt + 1) * T, i0:i0 + T] = blk.T


def kernel(batch_points, W1, b1, W2, b2, W3, b3, Wm1, bm1, Wm2, bm2):
    B, N = _B, _N
    pT = jnp.transpose(batch_points, (0, 2, 1))
    b1r = b1.reshape(1, _HID)
    b2r = b2.reshape(1, _HID)
    b3r = b3.reshape(1, _HID)
    bm1r = bm1.reshape(1, _HID)
    Wm2T = Wm2.reshape(1, _HID)
    bm2r = bm2.reshape(1, 1)

    rep = lambda shape: pl.BlockSpec(shape, lambda g: (0,) * len(shape))
    out = pl.pallas_call(
        _edge_predictor_kernel,
        grid=(B,),
        in_specs=[
            pl.BlockSpec((1, N, 2), lambda g: (g, 0, 0)),
            pl.BlockSpec((1, 2, N), lambda g: (g, 0, 0)),
            rep((2, _HID)), rep((1, _HID)),
            rep((_HID, _HID)), rep((1, _HID)),
            rep((_HID, _HID)), rep((1, _HID)),
            rep((2 * _HID, _HID)), rep((1, _HID)),
            rep((1, _HID)), rep((1, 1)),
        ],
        out_specs=pl.BlockSpec((1, N, N), lambda g: (g, 0, 0)),
        out_shape=jax.ShapeDtypeStruct((B, N, N), jnp.float32),
        compiler_params=pltpu.CompilerParams(
            dimension_semantics=("parallel",)),
    )(batch_points, pT, W1, b1r, W2, b2r, W3, b3r, Wm1, bm1r, Wm2T, bm2r)
    return out
